# baseline (speedup 1.0000x reference)
import jax
import jax.numpy as jnp
from jax import lax
from jax.experimental import pallas as pl
from jax.experimental.pallas import tpu as pltpu
from jax.experimental.pallas import tpu_sc as plsc

_info = plsc.get_sparse_core_info()
_NC, _NS, _L = _info.num_cores, _info.num_subcores, _info.num_lanes
_NW = _NC * _NS
_B = 16384
_V = 1000000
_D = 64
_BPW = _B // _NW
_CHUNK = 128
_NH = 2
_HPW = _BPW // _NH
_NG = _HPW // _L
_BITREV = [0, 8, 4, 12, 2, 10, 6, 14, 1, 9, 5, 13, 3, 11, 7, 15]


def _dots_body(cw_hbm, tw_hbm, embv_hbm, embu_hbm,
               dot_hbm,
               cw_v, tw_v, cemb, temb, rbuf, dot_v, sem):
    wid = lax.axis_index("s") * _NC + lax.axis_index("c")
    base = pl.multiple_of(wid * _BPW, _BPW)

    pltpu.sync_copy(cw_hbm.at[pl.ds(base, _BPW)], cw_v)
    pltpu.sync_copy(tw_hbm.at[pl.ds(base, _BPW)], tw_v)

    lane = lax.iota(jnp.int32, _L)
    masks = {h: (lane & h) == 0 for h in (8, 4, 2, 1)}

    for half in range(_NH):
        emb_copies = []
        for c in range(_HPW // _CHUNK):
            si = pl.ds(half * _HPW + c * _CHUNK, _CHUNK)
            so = pl.ds(c * _CHUNK, _CHUNK)
            emb_copies.append(pltpu.async_copy(embv_hbm.at[cw_v.at[si]], cemb.at[so], sem))
            emb_copies.append(pltpu.async_copy(embu_hbm.at[tw_v.at[si]], temb.at[so], sem))
        for cp in emb_copies:
            cp.wait()

        def group(g, carry):
            b0 = pl.multiple_of(g * _L, _L)
            vals = []
            for i, j in enumerate(_BITREV):
                b = b0 + j
                p = cemb[b, pl.ds(0, _L)] * temb[b, pl.ds(0, _L)]
                for k in range(1, _D // _L):
                    p = p + cemb[b, pl.ds(k * _L, _L)] * temb[b, pl.ds(k * _L, _L)]
                sb = 8 + 32 * i
                rbuf[pl.ds(sb, _L)] = p
                vals.append((p, sb))
            slot = _L
            for h in (8, 4, 2, 1):
                m = masks[h]
                nxt = []
                for t in range(len(vals) // 2):
                    (av, ab), (bv, bb) = vals[2 * t], vals[2 * t + 1]
                    a_rot = rbuf[pl.ds(ab + h, _L)]
                    b_rot = rbuf[pl.ds(bb - h, _L)]
                    cv = jnp.where(m, av + a_rot, bv + b_rot)
                    sb2 = -1
                    if h > 1:
                        sb2 = 8 + 32 * slot
                        slot += 1
                        rbuf[pl.ds(sb2, _L)] = cv
                    nxt.append((cv, sb2))
                vals = nxt
            e0 = pl.multiple_of(half * _HPW + g * _L, _L)
            dot_v[pl.ds(e0, _L)] = vals[0][0]
            return carry

        lax.fori_loop(0, _NG, group, jnp.int32(0))

    pltpu.sync_copy(dot_v, dot_hbm.at[pl.ds(base, _BPW)])


_glove_dots = pl.kernel(
    _dots_body,
    out_type=jax.ShapeDtypeStruct((_B,), jnp.float32),
    mesh=plsc.VectorSubcoreMesh(core_axis_name="c", subcore_axis_name="s"),
    compiler_params=pltpu.CompilerParams(use_tc_tiling_on_sc=False),
    scratch_types=[
        pltpu.VMEM((_BPW,), jnp.int32),
        pltpu.VMEM((_BPW,), jnp.int32),
        pltpu.VMEM((_HPW, _D), jnp.float32),
        pltpu.VMEM((_HPW, _D), jnp.float32),
        pltpu.VMEM((1024,), jnp.float32),
        pltpu.VMEM((_BPW,), jnp.float32),
        pltpu.SemaphoreType.DMA,
    ],
)


def _bias_body(cw_hbm, tw_hbm, vb_hbm,
               cbo_hbm, tbo_hbm,
               cw_v, tw_v, cb_v, tb_v, sem):
    wid = lax.axis_index("s") * _NC + lax.axis_index("c")
    base = pl.multiple_of(wid * _BPW, _BPW)
    pltpu.sync_copy(cw_hbm.at[pl.ds(base, _BPW)], cw_v)
    pltpu.sync_copy(tw_hbm.at[pl.ds(base, _BPW)], tw_v)
    copies = []
    for c in range(_BPW // _CHUNK):
        s = pl.ds(c * _CHUNK, _CHUNK)
        copies.append(pltpu.async_copy(vb_hbm.at[cw_v.at[s]], cb_v.at[s], sem))
        copies.append(pltpu.async_copy(vb_hbm.at[tw_v.at[s]], tb_v.at[s], sem))
    for cp in copies:
        cp.wait()
    pltpu.sync_copy(cb_v, cbo_hbm.at[pl.ds(base, _BPW)])
    pltpu.sync_copy(tb_v, tbo_hbm.at[pl.ds(base, _BPW)])


_glove_biases = pl.kernel(
    _bias_body,
    out_type=(
        jax.ShapeDtypeStruct((_B, 1), jnp.float32),
        jax.ShapeDtypeStruct((_B, 1), jnp.float32),
    ),
    mesh=plsc.VectorSubcoreMesh(core_axis_name="c", subcore_axis_name="s"),
    compiler_params=pltpu.CompilerParams(use_tc_tiling_on_sc=False),
    scratch_types=[
        pltpu.VMEM((_BPW,), jnp.int32),
        pltpu.VMEM((_BPW,), jnp.int32),
        pltpu.VMEM((_BPW, 1), jnp.float32),
        pltpu.VMEM((_BPW, 1), jnp.float32),
        pltpu.SemaphoreType.DMA,
    ],
)


def _loss_body(dot_ref, cb_ref, tb_ref, cooc_ref, wt_ref, o_ref):
    err = (dot_ref[...] + cb_ref[...] + tb_ref[...] - cooc_ref[...])
    o_ref[...] = jnp.sum(wt_ref[...] * err * err, keepdims=True)


def kernel(center_words, target_words, coocs, weights, emb_v, emb_u, v_bias,
           u_bias):
    del u_bias
    cw = center_words.reshape(_B)
    tw = target_words.reshape(_B)
    dots = _glove_dots(cw, tw, emb_v, emb_u)
    cb, tb = _glove_biases(cw, tw, v_bias)
    total = pl.pallas_call(
        _loss_body,
        out_shape=jax.ShapeDtypeStruct((1, 1), jnp.float32),
    )(dots.reshape(_B, 1), cb, tb, coocs, weights)
    return total[0, 0]


# speedup vs baseline: 1.7320x; 1.7320x over previous
import jax
import jax.numpy as jnp
from jax import lax
from jax.experimental import pallas as pl
from jax.experimental.pallas import tpu as pltpu
from jax.experimental.pallas import tpu_sc as plsc

_info = plsc.get_sparse_core_info()
_NC, _NS, _L = _info.num_cores, _info.num_subcores, _info.num_lanes
_NW = _NC * _NS
_B = 16384
_V = 1000000
_D = 64
_DP = 128
_BPW = _B // _NW
_CHUNK = 128
_NH = 2
_HPW = _BPW // _NH
_NG = _HPW // _L
_BITREV = [0, 8, 4, 12, 2, 10, 6, 14, 1, 9, 5, 13, 3, 11, 7, 15]


def _butterfly(rbuf, vals, masks, slot0):
    slot = slot0
    for h in (8, 4, 2, 1):
        m = masks[h]
        nxt = []
        for t in range(len(vals) // 2):
            (av, ab), (bv, bb) = vals[2 * t], vals[2 * t + 1]
            a_rot = rbuf[pl.ds(ab + h, _L)]
            b_rot = rbuf[pl.ds(bb - h, _L)]
            cv = jnp.where(m, av + a_rot, bv + b_rot)
            sb = -1
            if h > 1:
                sb = 8 + 32 * slot
                slot += 1
                rbuf[pl.ds(sb, _L)] = cv
            nxt.append((cv, sb))
        vals = nxt
    return vals[0][0]


def _glove_body(cw_hbm, tw_hbm, embv_hbm, embu_hbm, vb_hbm,
                dot_hbm, cbo_hbm, tbo_hbm,
                cw_v, tw_v, hc_v, ht_v, cb_v, tb_v, cemb, temb,
                rbuf, dot_v, sem):
    wid = lax.axis_index("s") * _NC + lax.axis_index("c")
    base = pl.multiple_of(wid * _BPW, _BPW)

    pltpu.sync_copy(cw_hbm.at[pl.ds(base, _BPW)], cw_v)
    pltpu.sync_copy(tw_hbm.at[pl.ds(base, _BPW)], tw_v)

    for c in range(_BPW // _L):
        s = pl.ds(c * _L, _L)
        hc_v[s] = lax.shift_right_logical(cw_v[s], 1)
        ht_v[s] = lax.shift_right_logical(tw_v[s], 1)

    bias_copies = []
    for c in range(_BPW // _CHUNK):
        s = pl.ds(c * _CHUNK, _CHUNK)
        bias_copies.append(pltpu.async_copy(vb_hbm.at[cw_v.at[s]], cb_v.at[s], sem))
        bias_copies.append(pltpu.async_copy(vb_hbm.at[tw_v.at[s]], tb_v.at[s], sem))

    lane = lax.iota(jnp.int32, _L)
    masks = {h: (lane & h) == 0 for h in (8, 4, 2, 1)}

    for half in range(_NH):
        emb_copies = []
        for c in range(_HPW // _CHUNK):
            si = pl.ds(half * _HPW + c * _CHUNK, _CHUNK)
            so = pl.ds(c * _CHUNK, _CHUNK)
            emb_copies.append(pltpu.async_copy(embv_hbm.at[hc_v.at[si]], cemb.at[so], sem))
            emb_copies.append(pltpu.async_copy(embu_hbm.at[ht_v.at[si]], temb.at[so], sem))
        for cp in emb_copies:
            cp.wait()

        def group(g, carry):
            b0 = pl.multiple_of(g * _L, _L)
            e0 = pl.multiple_of(half * _HPW + g * _L, _L)
            combo_vals = {(ch, th): [] for ch in (0, 1) for th in (0, 1)}
            for i, j in enumerate(_BITREV):
                b = b0 + j
                cc = [cemb[b, pl.ds(k * _L, _L)] for k in range(_DP // _L)]
                tt = [temb[b, pl.ds(k * _L, _L)] for k in range(_DP // _L)]
                nk = _D // _L
                for ci, (ch, th) in enumerate(combo_vals):
                    p = cc[ch * nk] * tt[th * nk]
                    for k in range(1, nk):
                        p = p + cc[ch * nk + k] * tt[th * nk + k]
                    sb = 8 + 32 * (i + 16 * ci)
                    rbuf[pl.ds(sb, _L)] = p
                    combo_vals[(ch, th)].append((p, sb))
            dots = {}
            for ci, key in enumerate(combo_vals):
                dots[key] = _butterfly(rbuf, combo_vals[key], masks,
                                       64 + 15 * ci)
            codd = (cw_v[pl.ds(e0, _L)] & 1) == 1
            todd = (tw_v[pl.ds(e0, _L)] & 1) == 1
            dotv = jnp.where(
                codd,
                jnp.where(todd, dots[(1, 1)], dots[(1, 0)]),
                jnp.where(todd, dots[(0, 1)], dots[(0, 0)]),
            )
            dot_v[pl.ds(e0, _L)] = dotv
            return carry

        lax.fori_loop(0, _NG, group, jnp.int32(0))

    for cp in bias_copies:
        cp.wait()
    pltpu.sync_copy(dot_v, dot_hbm.at[pl.ds(base, _BPW)])
    pltpu.sync_copy(cb_v, cbo_hbm.at[pl.ds(base, _BPW)])
    pltpu.sync_copy(tb_v, tbo_hbm.at[pl.ds(base, _BPW)])


_glove_gather = pl.kernel(
    _glove_body,
    out_type=(
        jax.ShapeDtypeStruct((_B,), jnp.float32),
        jax.ShapeDtypeStruct((_B,), jnp.float32),
        jax.ShapeDtypeStruct((_B,), jnp.float32),
    ),
    mesh=plsc.VectorSubcoreMesh(core_axis_name="c", subcore_axis_name="s"),
    compiler_params=pltpu.CompilerParams(use_tc_tiling_on_sc=True),
    scratch_types=[
        pltpu.VMEM((_BPW,), jnp.int32),
        pltpu.VMEM((_BPW,), jnp.int32),
        pltpu.VMEM((_BPW,), jnp.int32),
        pltpu.VMEM((_BPW,), jnp.int32),
        pltpu.VMEM((_BPW,), jnp.float32),
        pltpu.VMEM((_BPW,), jnp.float32),
        pltpu.VMEM((_HPW, _DP), jnp.float32),
        pltpu.VMEM((_HPW, _DP), jnp.float32),
        pltpu.VMEM((4096,), jnp.float32),
        pltpu.VMEM((_BPW,), jnp.float32),
        pltpu.SemaphoreType.DMA,
    ],
)


def _loss_body(dot_ref, cb_ref, tb_ref, cooc_ref, wt_ref, o_ref):
    err = (dot_ref[...] + cb_ref[...] + tb_ref[...] - cooc_ref[...])
    o_ref[...] = jnp.sum(wt_ref[...] * err * err, keepdims=True)


def kernel(center_words, target_words, coocs, weights, emb_v, emb_u, v_bias,
           u_bias):
    del u_bias
    cw = center_words.reshape(_B)
    tw = target_words.reshape(_B)
    dots, cb, tb = _glove_gather(cw, tw,
                                 emb_v.reshape(_V // 2, _DP),
                                 emb_u.reshape(_V // 2, _DP),
                                 v_bias.reshape(_V))
    total = pl.pallas_call(
        _loss_body,
        out_shape=jax.ShapeDtypeStruct((1, 1), jnp.float32),
    )(dots.reshape(_B, 1), cb.reshape(_B, 1), tb.reshape(_B, 1),
      coocs, weights)
    return total[0, 0]


# speedup vs baseline: 1.7450x; 1.0075x over previous
import jax
import jax.numpy as jnp
from jax import lax
from jax.experimental import pallas as pl
from jax.experimental.pallas import tpu as pltpu
from jax.experimental.pallas import tpu_sc as plsc

_info = plsc.get_sparse_core_info()
_NC, _NS, _L = _info.num_cores, _info.num_subcores, _info.num_lanes
_NW = _NC * _NS
_B = 16384
_V = 1000000
_D = 64
_BPW = _B // _NW
_CHUNK = 128
_NH = 2
_HPW = _BPW // _NH
_NG = _HPW // _L
_BITREV = [0, 8, 4, 12, 2, 10, 6, 14, 1, 9, 5, 13, 3, 11, 7, 15]


def _glove_body(cw_hbm, tw_hbm, embv_hbm, embu_hbm, vbt_hbm,
                dot_hbm, cbo_hbm, tbo_hbm,
                cw_v, tw_v, cb_v, tb_v, cemb, temb, rbuf, dot_v, sem):
    wid = lax.axis_index("s") * _NC + lax.axis_index("c")
    base = pl.multiple_of(wid * _BPW, _BPW)

    pltpu.sync_copy(cw_hbm.at[pl.ds(base, _BPW)], cw_v)
    pltpu.sync_copy(tw_hbm.at[pl.ds(base, _BPW)], tw_v)

    vb1 = vbt_hbm.at[0]
    bias_copies = []
    for c in range(_BPW // _CHUNK):
        s = pl.ds(c * _CHUNK, _CHUNK)
        bias_copies.append(pltpu.async_copy(vb1.at[cw_v.at[s]], cb_v.at[s], sem))
        bias_copies.append(pltpu.async_copy(vb1.at[tw_v.at[s]], tb_v.at[s], sem))

    lane = lax.iota(jnp.int32, _L)
    masks = {h: (lane & h) == 0 for h in (8, 4, 2, 1)}

    for half in range(_NH):
        emb_copies = []
        for c in range(_HPW // _CHUNK):
            si = pl.ds(half * _HPW + c * _CHUNK, _CHUNK)
            so = pl.ds(c * _CHUNK, _CHUNK)
            emb_copies.append(pltpu.async_copy(embv_hbm.at[cw_v.at[si]], cemb.at[so], sem))
            emb_copies.append(pltpu.async_copy(embu_hbm.at[tw_v.at[si]], temb.at[so], sem))
        for cp in emb_copies:
            cp.wait()

        def group(g, carry):
            b0 = pl.multiple_of(g * _L, _L)
            vals = []
            for i, j in enumerate(_BITREV):
                b = b0 + j
                p = cemb[b, pl.ds(0, _L)] * temb[b, pl.ds(0, _L)]
                for k in range(1, _D // _L):
                    p = p + cemb[b, pl.ds(k * _L, _L)] * temb[b, pl.ds(k * _L, _L)]
                sb = 8 + 32 * i
                rbuf[pl.ds(sb, _L)] = p
                vals.append((p, sb))
            slot = _L
            for h in (8, 4, 2, 1):
                m = masks[h]
                nxt = []
                for t in range(len(vals) // 2):
                    (av, ab), (bv, bb) = vals[2 * t], vals[2 * t + 1]
                    a_rot = rbuf[pl.ds(ab + h, _L)]
                    b_rot = rbuf[pl.ds(bb - h, _L)]
                    cv = jnp.where(m, av + a_rot, bv + b_rot)
                    sb2 = -1
                    if h > 1:
                        sb2 = 8 + 32 * slot
                        slot += 1
                        rbuf[pl.ds(sb2, _L)] = cv
                    nxt.append((cv, sb2))
                vals = nxt
            e0 = pl.multiple_of(half * _HPW + g * _L, _L)
            dot_v[pl.ds(e0, _L)] = vals[0][0]
            return carry

        lax.fori_loop(0, _NG, group, jnp.int32(0))

    for cp in bias_copies:
        cp.wait()
    pltpu.sync_copy(dot_v, dot_hbm.at[pl.ds(base, _BPW)])
    pltpu.sync_copy(cb_v, cbo_hbm.at[pl.ds(base, _BPW)])
    pltpu.sync_copy(tb_v, tbo_hbm.at[pl.ds(base, _BPW)])


_glove_gather = pl.kernel(
    _glove_body,
    out_type=(
        jax.ShapeDtypeStruct((_B,), jnp.float32),
        jax.ShapeDtypeStruct((_B,), jnp.float32),
        jax.ShapeDtypeStruct((_B,), jnp.float32),
    ),
    mesh=plsc.VectorSubcoreMesh(core_axis_name="c", subcore_axis_name="s"),
    compiler_params=pltpu.CompilerParams(use_tc_tiling_on_sc=False),
    scratch_types=[
        pltpu.VMEM((_BPW,), jnp.int32),
        pltpu.VMEM((_BPW,), jnp.int32),
        pltpu.VMEM((_BPW,), jnp.float32),
        pltpu.VMEM((_BPW,), jnp.float32),
        pltpu.VMEM((_HPW, _D), jnp.float32),
        pltpu.VMEM((_HPW, _D), jnp.float32),
        pltpu.VMEM((1024,), jnp.float32),
        pltpu.VMEM((_BPW,), jnp.float32),
        pltpu.SemaphoreType.DMA,
    ],
)


def _loss_body(dot_ref, cb_ref, tb_ref, cooc_ref, wt_ref, o_ref):
    err = (dot_ref[...] + cb_ref[...] + tb_ref[...] - cooc_ref[...])
    o_ref[...] = jnp.sum(wt_ref[...] * err * err, keepdims=True)


def kernel(center_words, target_words, coocs, weights, emb_v, emb_u, v_bias,
           u_bias):
    del u_bias
    cw = center_words.reshape(_B)
    tw = target_words.reshape(_B)
    dots, cb, tb = _glove_gather(cw, tw, emb_v, emb_u, v_bias.T)
    total = pl.pallas_call(
        _loss_body,
        out_shape=jax.ShapeDtypeStruct((1, 1), jnp.float32),
    )(dots.reshape(_B, 1), cb.reshape(_B, 1), tb.reshape(_B, 1),
      coocs, weights)
    return total[0, 0]


# speedup vs baseline: 1.7926x; 1.0273x over previous
import jax
import jax.numpy as jnp
from jax import lax
from jax.experimental import pallas as pl
from jax.experimental.pallas import tpu as pltpu
from jax.experimental.pallas import tpu_sc as plsc

_info = plsc.get_sparse_core_info()
_NC, _NS, _L = _info.num_cores, _info.num_subcores, _info.num_lanes
_NW = _NC * _NS
_B = 16384
_V = 1000000
_D = 64
_BPW = _B // _NW
_CHUNK = 128
_NH = 2
_HPW = _BPW // _NH
_NG = _HPW // _L
_BITREV = [0, 8, 4, 12, 2, 10, 6, 14, 1, 9, 5, 13, 3, 11, 7, 15]


def _glove_body(cw_hbm, tw_hbm, cooc_hbm, wt_hbm, embv_hbm, embu_hbm,
                vbt_hbm,
                out_hbm,
                cw_v, tw_v, cooc_v, wt_v, cb_v, tb_v, cemb, temb,
                rbuf, acc_v, sem):
    wid = lax.axis_index("s") * _NC + lax.axis_index("c")
    base = pl.multiple_of(wid * _BPW, _BPW)

    pltpu.sync_copy(cw_hbm.at[pl.ds(base, _BPW)], cw_v)
    pltpu.sync_copy(tw_hbm.at[pl.ds(base, _BPW)], tw_v)
    pltpu.sync_copy(cooc_hbm.at[pl.ds(base, _BPW)], cooc_v)
    pltpu.sync_copy(wt_hbm.at[pl.ds(base, _BPW)], wt_v)

    vb1 = vbt_hbm.at[0]
    bias_copies = []
    for c in range(_BPW // _CHUNK):
        s = pl.ds(c * _CHUNK, _CHUNK)
        bias_copies.append(pltpu.async_copy(vb1.at[cw_v.at[s]], cb_v.at[s], sem))
        bias_copies.append(pltpu.async_copy(vb1.at[tw_v.at[s]], tb_v.at[s], sem))

    lane = lax.iota(jnp.int32, _L)
    masks = {h: (lane & h) == 0 for h in (8, 4, 2, 1)}

    acc = jnp.zeros((_L,), jnp.float32)
    for half in range(_NH):
        emb_copies = []
        for c in range(_HPW // _CHUNK):
            si = pl.ds(half * _HPW + c * _CHUNK, _CHUNK)
            so = pl.ds(c * _CHUNK, _CHUNK)
            emb_copies.append(pltpu.async_copy(embv_hbm.at[cw_v.at[si]], cemb.at[so], sem))
            emb_copies.append(pltpu.async_copy(embu_hbm.at[tw_v.at[si]], temb.at[so], sem))
        for cp in emb_copies:
            cp.wait()
        if half == 0:
            for cp in bias_copies:
                cp.wait()

        def group(g, acc):
            b0 = pl.multiple_of(g * _L, _L)
            vals = []
            for i, j in enumerate(_BITREV):
                b = b0 + j
                p = cemb[b, pl.ds(0, _L)] * temb[b, pl.ds(0, _L)]
                for k in range(1, _D // _L):
                    p = p + cemb[b, pl.ds(k * _L, _L)] * temb[b, pl.ds(k * _L, _L)]
                sb = 8 + 32 * i
                rbuf[pl.ds(sb, _L)] = p
                vals.append((p, sb))
            slot = _L
            for h in (8, 4, 2, 1):
                m = masks[h]
                nxt = []
                for t in range(len(vals) // 2):
                    (av, ab), (bv, bb) = vals[2 * t], vals[2 * t + 1]
                    a_rot = rbuf[pl.ds(ab + h, _L)]
                    b_rot = rbuf[pl.ds(bb - h, _L)]
                    cv = jnp.where(m, av + a_rot, bv + b_rot)
                    sb2 = -1
                    if h > 1:
                        sb2 = 8 + 32 * slot
                        slot += 1
                        rbuf[pl.ds(sb2, _L)] = cv
                    nxt.append((cv, sb2))
                vals = nxt
            dotv = vals[0][0]
            e0 = pl.multiple_of(half * _HPW + g * _L, _L)
            cb = cb_v[pl.ds(e0, _L)]
            tb = tb_v[pl.ds(e0, _L)]
            cooc = cooc_v[pl.ds(e0, _L)]
            wt = wt_v[pl.ds(e0, _L)]
            err = dotv + cb + tb - cooc
            return acc + wt * err * err

        acc = lax.fori_loop(0, _NG, group, acc)

    acc_v[...] = acc
    pltpu.sync_copy(acc_v, out_hbm.at[wid])


_glove_partials = pl.kernel(
    _glove_body,
    out_type=jax.ShapeDtypeStruct((_NW, _L), jnp.float32),
    mesh=plsc.VectorSubcoreMesh(core_axis_name="c", subcore_axis_name="s"),
    compiler_params=pltpu.CompilerParams(use_tc_tiling_on_sc=False),
    scratch_types=[
        pltpu.VMEM((_BPW,), jnp.int32),
        pltpu.VMEM((_BPW,), jnp.int32),
        pltpu.VMEM((_BPW,), jnp.float32),
        pltpu.VMEM((_BPW,), jnp.float32),
        pltpu.VMEM((_BPW,), jnp.float32),
        pltpu.VMEM((_BPW,), jnp.float32),
        pltpu.VMEM((_HPW, _D), jnp.float32),
        pltpu.VMEM((_HPW, _D), jnp.float32),
        pltpu.VMEM((1024,), jnp.float32),
        pltpu.VMEM((_L,), jnp.float32),
        pltpu.SemaphoreType.DMA,
    ],
)


def _sum_body(x_ref, o_ref):
    o_ref[...] = jnp.sum(x_ref[...], keepdims=True)


def kernel(center_words, target_words, coocs, weights, emb_v, emb_u, v_bias,
           u_bias):
    del u_bias
    cw = center_words.reshape(_B)
    tw = target_words.reshape(_B)
    cooc = coocs.reshape(_B)
    wt = weights.reshape(_B)
    partials = _glove_partials(cw, tw, cooc, wt, emb_v, emb_u, v_bias.T)
    total = pl.pallas_call(
        _sum_body,
        out_shape=jax.ShapeDtypeStruct((1, 1), jnp.float32),
    )(partials)
    return total[0, 0]
